# Initial kernel scaffold; baseline (speedup 1.0000x reference)
import jax
import jax.numpy as jnp
from jax.experimental import pallas as pl


def kernel(x, edge_index, edge_attr, u, batch, W1, b1, W2, b2, W3, b3):
    raise NotImplementedError("write your pallas kernel here")


# speedup vs baseline: 5.8469x; 5.8469x over previous
import jax
import jax.numpy as jnp
from jax import lax
from jax.experimental import pallas as pl

N_NODES = 10000
N_GRAPHS = 64
HIDDEN = 512


def _body(x_ref, b_ref, u_ref, w1u_ref, w1x_ref, b1_ref, w2_ref, b2_ref,
          w3_ref, b3_ref, out_ref):
    batch = b_ref[...]
    seg_iota = lax.broadcasted_iota(jnp.int32, (N_NODES, N_GRAPHS), 1)
    onehot = (batch == seg_iota).astype(jnp.float32)
    sums = lax.dot_general(
        onehot, x_ref[...], (((0,), (0,)), ((), ())),
        preferred_element_type=jnp.float32,
        precision=lax.Precision.HIGHEST,
    )
    cnt = jnp.sum(onehot, axis=0)[:, None]
    mean = sums / jnp.maximum(cnt, 1.0)
    h = (u_ref[...] @ w1u_ref[...]
         + lax.dot_general(mean, w1x_ref[...], (((1,), (0,)), ((), ())),
                           preferred_element_type=jnp.float32,
                           precision=lax.Precision.HIGHEST)
         + b1_ref[...])
    h = jnp.maximum(h, 0.0)
    h = lax.dot_general(h, w2_ref[...], (((1,), (0,)), ((), ())),
                        preferred_element_type=jnp.float32,
                        precision=lax.Precision.HIGHEST) + b2_ref[...]
    h = jnp.maximum(h, 0.0)
    out_ref[...] = lax.dot_general(h, w3_ref[...], (((1,), (0,)), ((), ())),
                                   preferred_element_type=jnp.float32,
                                   precision=lax.Precision.HIGHEST) + b3_ref[...]


def kernel(x, edge_index, edge_attr, u, batch, W1, b1, W2, b2, W3, b3):
    del edge_index, edge_attr
    batch32 = batch.astype(jnp.int32).reshape(N_NODES, 1)
    u2 = u.reshape(N_GRAPHS, 6)
    W1u = W1[:6]
    W1x = W1[6:]
    return pl.pallas_call(
        _body,
        out_shape=jax.ShapeDtypeStruct((N_GRAPHS, 128), jnp.float32),
    )(x, batch32, u2, W1u, W1x, b1.reshape(1, HIDDEN), W2,
      b2.reshape(1, HIDDEN), W3, b3.reshape(1, 128))
